# baseline (speedup 1.0000x reference)
import functools

import jax
import jax.numpy as jnp
from jax import lax
from jax.experimental import pallas as pl
from jax.experimental.pallas import tpu as pltpu
from jax.experimental.pallas import tpu_sc as plsc

N = 10000
E = 320000
D = 128

NC = 2
NS = 16
NW = NC * NS
L = 16

EPT = 10240
E_PAD = NW * EPT

C = 80
CH = EPT // C
NB = 4
ND = 8
GL = 3
FL = 4

NPAD = 10240
RPS = NPAD // NS
RC = 128

_mesh = plsc.VectorSubcoreMesh(
    core_axis_name="c", subcore_axis_name="s", num_cores=NC, num_subcores=NS)


HR = NPAD // 128


@functools.partial(
    pl.kernel,
    out_type=jax.ShapeDtypeStruct((NC, HR, 128), jnp.float32),
    mesh=_mesh,
    compiler_params=pltpu.CompilerParams(needs_layout_passes=False),
    scratch_types=[
        pltpu.VMEM((HR, 128), jnp.float32),
        pltpu.VMEM((EPT,), jnp.int32),
        pltpu.VMEM((HR,), jnp.int32),
        pltpu.VMEM_SHARED((HR, 128), jnp.float32),
    ],
)
def _deg_kernel(dst_hbm, out_hbm, hist, didx, rowidx, acc):
    cid = lax.axis_index("c")
    sid = lax.axis_index("s")
    wid = cid * NS + sid

    zeros = jnp.zeros((L,), jnp.float32)

    def _zero(i, _):
        hist[i // (128 // L), pl.ds((i % (128 // L)) * L, L)] = zeros
        return 0

    lax.fori_loop(0, NPAD // L, _zero, 0)

    for i in range(HR // L):
        rowidx[pl.ds(i * L, L)] = lax.iota(jnp.int32, L) + (i * L)

    @pl.when(sid == 0)
    def _():
        pltpu.sync_copy(hist, acc)

    pltpu.sync_copy(dst_hbm.at[pl.ds(wid * EPT, EPT)], didx)

    ones = jnp.ones((L,), jnp.float32)

    def _accum(i, _):
        d = didx[pl.ds(i * L, L)]
        plsc.addupdate_scatter(hist, [d >> 7, d & 127], ones)
        return 0

    lax.fori_loop(0, EPT // L, _accum, 0)

    plsc.subcore_barrier()
    pltpu.sync_copy(hist, acc.at[rowidx], add=True)
    plsc.subcore_barrier()
    @pl.when(sid < HR // 8)
    def _():
        pltpu.sync_copy(acc.at[pl.ds(sid * 8, 8)],
                        out_hbm.at[cid, pl.ds(sid * 8, 8)])


@functools.partial(
    pl.kernel,
    out_type=jax.ShapeDtypeStruct((NC, NPAD, D), jnp.float32),
    mesh=_mesh,
    compiler_params=pltpu.CompilerParams(needs_layout_passes=False),
    scratch_types=[
        [pltpu.VMEM((C,), jnp.int32)] * NB,
        [pltpu.VMEM((C,), jnp.int32)] * ND,
        pltpu.VMEM((NB, C, D), jnp.float32),
        pltpu.VMEM_SHARED((NPAD, D), jnp.float32),
        [pltpu.SemaphoreType.DMA] * NB,
        [pltpu.SemaphoreType.DMA] * ND,
        [pltpu.SemaphoreType.DMA] * NB,
        [pltpu.SemaphoreType.DMA] * NB,
    ],
)
def _agg_kernel(y_hbm, src_hbm, dst_hbm, out_hbm,
                sidxs, didxs, rows, acc, fsems, dsems, gsems, ssems):
    cid = lax.axis_index("c")
    sid = lax.axis_index("s")
    wid = cid * NS + sid
    ebase = wid * EPT

    zeros = jnp.zeros((L,), jnp.float32)

    def _zero(i, _):
        rows[0, i // (D // L), pl.ds((i % (D // L)) * L, L)] = zeros
        return 0

    lax.fori_loop(0, RC * D // L, _zero, 0)
    for k in range(RPS // RC):
        pltpu.sync_copy(rows.at[0, pl.ds(0, RC)],
                        acc.at[pl.ds(sid * RPS + k * RC, RC)])
    plsc.subcore_barrier()

    def _ifetch(j, bs, bd):
        pltpu.async_copy(src_hbm.at[pl.ds(ebase + j * C, C)], sidxs[bs],
                         fsems[bs])
        pltpu.async_copy(dst_hbm.at[pl.ds(ebase + j * C, C)], didxs[bd],
                         dsems[bd])

    def _gather(j, b):
        pltpu.make_async_copy(src_hbm.at[pl.ds(0, C)], sidxs[b],
                              fsems[b]).wait()
        pltpu.async_copy(y_hbm.at[sidxs[b]], rows.at[b], gsems[b])

    for j in range(FL):
        _ifetch(j, j % NB, j % ND)
    for j in range(GL):
        _gather(j, j % NB)

    def _group(g, _):
        for b8 in range(ND):
            j = g * ND + b8
            b = b8 % NB

            pltpu.make_async_copy(y_hbm.at[sidxs[b]], rows.at[b],
                                  gsems[b]).wait()
            pltpu.make_async_copy(dst_hbm.at[pl.ds(0, C)], didxs[b8],
                                  dsems[b8]).wait()
            pltpu.async_copy(rows.at[b], acc.at[didxs[b8]], ssems[b],
                             add=True)

            @pl.when(j + FL < CH)
            def _():
                _ifetch(j + FL, b, (b8 + FL) % ND)

            jg = j + GL
            bg = (b + GL) % NB

            @pl.when(jg < CH)
            def _():
                @pl.when(jg >= NB)
                def _():
                    pltpu.make_async_copy(
                        rows.at[bg], acc.at[didxs[0]], ssems[bg]).wait()

                _gather(jg, bg)

        return 0

    lax.fori_loop(0, CH // ND, _group, 0)

    for b in range(NB):
        pltpu.make_async_copy(rows.at[b], acc.at[didxs[0]], ssems[b]).wait()

    plsc.subcore_barrier()
    for k in range(RPS // RC):
        r = sid * RPS + k * RC
        pltpu.sync_copy(acc.at[pl.ds(r, RC)], out_hbm.at[cid, pl.ds(r, RC)])


_R = 1024


def _tc1_body(x_ref, w_ref, degp_ref, y_ref):
    deg = degp_ref[0, :] + degp_ref[1, :] + 1.0
    dinv = lax.rsqrt(deg)
    xw = jnp.dot(x_ref[...], w_ref[...], preferred_element_type=jnp.float32)
    y_ref[...] = xw * dinv[:, None]


def _tc2_body(qp_ref, y_ref, degp_ref, b_ref, o_ref):
    deg = degp_ref[0, :] + degp_ref[1, :] + 1.0
    dinv = lax.rsqrt(deg)
    s = qp_ref[0] + qp_ref[1] + y_ref[...]
    o_ref[...] = s * dinv[:, None] + b_ref[...]


def kernel(x, edge_index, W, b):
    src = edge_index[0]
    dst = edge_index[1]
    pad = E_PAD - E
    ar = jnp.arange(pad, dtype=jnp.int32)
    src_p = jnp.concatenate([src, ar % N])
    dst_p = jnp.concatenate([dst, N + (ar % (NPAD - N))])

    degp = _deg_kernel(dst_p).reshape(NC, NPAD)

    y = pl.pallas_call(
        _tc1_body,
        grid=(pl.cdiv(N, _R),),
        in_specs=[
            pl.BlockSpec((_R, D), lambda i: (i, 0)),
            pl.BlockSpec((D, D), lambda i: (0, 0)),
            pl.BlockSpec((NC, _R), lambda i: (0, i)),
        ],
        out_specs=pl.BlockSpec((_R, D), lambda i: (i, 0)),
        out_shape=jax.ShapeDtypeStruct((N, D), jnp.float32),
    )(x, W, degp)

    qp = _agg_kernel(y, src_p, dst_p)

    out = pl.pallas_call(
        _tc2_body,
        grid=(pl.cdiv(N, _R),),
        in_specs=[
            pl.BlockSpec((NC, _R, D), lambda i: (0, i, 0)),
            pl.BlockSpec((_R, D), lambda i: (i, 0)),
            pl.BlockSpec((NC, _R), lambda i: (0, i)),
            pl.BlockSpec((1, D), lambda i: (0, 0)),
        ],
        out_specs=pl.BlockSpec((_R, D), lambda i: (i, 0)),
        out_shape=jax.ShapeDtypeStruct((N, D), jnp.float32),
    )(qp, y, degp, b.reshape(1, D))

    return out


# speedup vs baseline: 1.0344x; 1.0344x over previous
import functools

import jax
import jax.numpy as jnp
from jax import lax
from jax.experimental import pallas as pl
from jax.experimental.pallas import tpu as pltpu
from jax.experimental.pallas import tpu_sc as plsc

N = 10000
E = 320000
D = 128

NC = 2
NS = 16
NW = NC * NS
L = 16

EPT = 10240
E_PAD = NW * EPT

C = 64
CH = EPT // C
NB = 5
ND = 10
GL = 4
FL = 5

NPAD = 10240
RPS = NPAD // NS
RC = 128

_mesh = plsc.VectorSubcoreMesh(
    core_axis_name="c", subcore_axis_name="s", num_cores=NC, num_subcores=NS)


HR = NPAD // 128


@functools.partial(
    pl.kernel,
    out_type=jax.ShapeDtypeStruct((NC, HR, 128), jnp.float32),
    mesh=_mesh,
    compiler_params=pltpu.CompilerParams(needs_layout_passes=False),
    scratch_types=[
        pltpu.VMEM((HR, 128), jnp.float32),
        pltpu.VMEM((EPT,), jnp.int32),
        pltpu.VMEM((HR,), jnp.int32),
        pltpu.VMEM_SHARED((HR, 128), jnp.float32),
    ],
)
def _deg_kernel(dst_hbm, out_hbm, hist, didx, rowidx, acc):
    cid = lax.axis_index("c")
    sid = lax.axis_index("s")
    wid = cid * NS + sid

    zeros = jnp.zeros((L,), jnp.float32)

    def _zero(i, _):
        hist[i // (128 // L), pl.ds((i % (128 // L)) * L, L)] = zeros
        return 0

    lax.fori_loop(0, NPAD // L, _zero, 0)

    for i in range(HR // L):
        rowidx[pl.ds(i * L, L)] = lax.iota(jnp.int32, L) + (i * L)

    @pl.when(sid == 0)
    def _():
        pltpu.sync_copy(hist, acc)

    pltpu.sync_copy(dst_hbm.at[pl.ds(wid * EPT, EPT)], didx)

    ones = jnp.ones((L,), jnp.float32)

    def _accum(i, _):
        d = didx[pl.ds(i * L, L)]
        plsc.addupdate_scatter(hist, [d >> 7, d & 127], ones)
        return 0

    lax.fori_loop(0, EPT // L, _accum, 0)

    plsc.subcore_barrier()
    pltpu.sync_copy(hist, acc.at[rowidx], add=True)
    plsc.subcore_barrier()
    @pl.when(sid < HR // 8)
    def _():
        pltpu.sync_copy(acc.at[pl.ds(sid * 8, 8)],
                        out_hbm.at[cid, pl.ds(sid * 8, 8)])


@functools.partial(
    pl.kernel,
    out_type=jax.ShapeDtypeStruct((NC, NPAD, D), jnp.float32),
    mesh=_mesh,
    compiler_params=pltpu.CompilerParams(needs_layout_passes=False),
    scratch_types=[
        [pltpu.VMEM((C,), jnp.int32)] * NB,
        [pltpu.VMEM((C,), jnp.int32)] * ND,
        pltpu.VMEM((NB, C, D), jnp.float32),
        pltpu.VMEM_SHARED((NPAD, D), jnp.float32),
        [pltpu.SemaphoreType.DMA] * NB,
        [pltpu.SemaphoreType.DMA] * ND,
        [pltpu.SemaphoreType.DMA] * NB,
        [pltpu.SemaphoreType.DMA] * NB,
    ],
)
def _agg_kernel(y_hbm, src_hbm, dst_hbm, out_hbm,
                sidxs, didxs, rows, acc, fsems, dsems, gsems, ssems):
    cid = lax.axis_index("c")
    sid = lax.axis_index("s")
    wid = cid * NS + sid
    ebase = wid * EPT

    zeros = jnp.zeros((L,), jnp.float32)

    def _zero(i, _):
        rows[0, i // (D // L), pl.ds((i % (D // L)) * L, L)] = zeros
        return 0

    lax.fori_loop(0, RC * D // L, _zero, 0)
    for k in range(RPS // RC):
        pltpu.sync_copy(rows.at[0, pl.ds(0, RC)],
                        acc.at[pl.ds(sid * RPS + k * RC, RC)])
    plsc.subcore_barrier()

    def _ifetch(j, bs, bd):
        pltpu.async_copy(src_hbm.at[pl.ds(ebase + j * C, C)], sidxs[bs],
                         fsems[bs])
        pltpu.async_copy(dst_hbm.at[pl.ds(ebase + j * C, C)], didxs[bd],
                         dsems[bd])

    def _gather(j, b):
        pltpu.make_async_copy(src_hbm.at[pl.ds(0, C)], sidxs[b],
                              fsems[b]).wait()
        pltpu.async_copy(y_hbm.at[sidxs[b]], rows.at[b], gsems[b])

    for j in range(FL):
        _ifetch(j, j % NB, j % ND)
    for j in range(GL):
        _gather(j, j % NB)

    def _group(g, _):
        for b8 in range(ND):
            j = g * ND + b8
            b = b8 % NB

            pltpu.make_async_copy(y_hbm.at[sidxs[b]], rows.at[b],
                                  gsems[b]).wait()
            pltpu.make_async_copy(dst_hbm.at[pl.ds(0, C)], didxs[b8],
                                  dsems[b8]).wait()
            pltpu.async_copy(rows.at[b], acc.at[didxs[b8]], ssems[b],
                             add=True)

            @pl.when(j + FL < CH)
            def _():
                _ifetch(j + FL, b, (b8 + FL) % ND)

            jg = j + GL
            bg = (b + GL) % NB

            @pl.when(jg < CH)
            def _():
                @pl.when(jg >= NB)
                def _():
                    pltpu.make_async_copy(
                        rows.at[bg], acc.at[didxs[0]], ssems[bg]).wait()

                _gather(jg, bg)

        return 0

    lax.fori_loop(0, CH // ND, _group, 0)

    for b in range(NB):
        pltpu.make_async_copy(rows.at[b], acc.at[didxs[0]], ssems[b]).wait()

    plsc.subcore_barrier()
    for k in range(RPS // RC):
        r = sid * RPS + k * RC
        pltpu.sync_copy(acc.at[pl.ds(r, RC)], out_hbm.at[cid, pl.ds(r, RC)])


_R = 1024


def _tc1_body(x_ref, w_ref, degp_ref, y_ref):
    deg = degp_ref[0, :] + degp_ref[1, :] + 1.0
    dinv = lax.rsqrt(deg)
    xw = jnp.dot(x_ref[...], w_ref[...], preferred_element_type=jnp.float32)
    y_ref[...] = xw * dinv[:, None]


def _tc2_body(qp_ref, y_ref, degp_ref, b_ref, o_ref):
    deg = degp_ref[0, :] + degp_ref[1, :] + 1.0
    dinv = lax.rsqrt(deg)
    s = qp_ref[0] + qp_ref[1] + y_ref[...]
    o_ref[...] = s * dinv[:, None] + b_ref[...]


def kernel(x, edge_index, W, b):
    src = edge_index[0]
    dst = edge_index[1]
    pad = E_PAD - E
    ar = jnp.arange(pad, dtype=jnp.int32)
    src_p = jnp.concatenate([src, ar % N])
    dst_p = jnp.concatenate([dst, N + (ar % (NPAD - N))])

    degp = _deg_kernel(dst_p).reshape(NC, NPAD)

    y = pl.pallas_call(
        _tc1_body,
        grid=(pl.cdiv(N, _R),),
        in_specs=[
            pl.BlockSpec((_R, D), lambda i: (i, 0)),
            pl.BlockSpec((D, D), lambda i: (0, 0)),
            pl.BlockSpec((NC, _R), lambda i: (0, i)),
        ],
        out_specs=pl.BlockSpec((_R, D), lambda i: (i, 0)),
        out_shape=jax.ShapeDtypeStruct((N, D), jnp.float32),
    )(x, W, degp)

    qp = _agg_kernel(y, src_p, dst_p)

    out = pl.pallas_call(
        _tc2_body,
        grid=(pl.cdiv(N, _R),),
        in_specs=[
            pl.BlockSpec((NC, _R, D), lambda i: (0, i, 0)),
            pl.BlockSpec((_R, D), lambda i: (i, 0)),
            pl.BlockSpec((NC, _R), lambda i: (0, i)),
            pl.BlockSpec((1, D), lambda i: (0, 0)),
        ],
        out_specs=pl.BlockSpec((_R, D), lambda i: (i, 0)),
        out_shape=jax.ShapeDtypeStruct((N, D), jnp.float32),
    )(qp, y, degp, b.reshape(1, D))

    return out
